# baseline (speedup 1.0000x reference)
import functools

import jax
import jax.numpy as jnp
from jax import lax
from jax.experimental import pallas as pl
from jax.experimental.pallas import tpu as pltpu
from jax.experimental.pallas import tpu_sc as plsc

E = 8
TOP_K = 2
D_MODEL = 1024
D_FF = 4096
T = 2048
C = 1024
EC = E * C
LANES = 128


def _router_body(x_ref, wr_ref, meta_ref):
    x = x_ref[...]
    wr = wr_ref[...]
    logits = jnp.dot(x, wr, preferred_element_type=jnp.float32)
    col = lax.broadcasted_iota(jnp.int32, (T, LANES), 1).astype(jnp.float32)
    real = col < E
    logits = jnp.where(real, logits, -1e30)
    mx = jnp.max(logits, axis=1, keepdims=True)
    ex = jnp.exp(logits - mx)
    probs = ex / jnp.sum(ex, axis=1, keepdims=True)
    psel = jnp.where(real, probs, -1.0)
    p0 = jnp.max(psel, axis=1, keepdims=True)
    e0 = jnp.min(jnp.where(psel >= p0, col, 128.0), axis=1, keepdims=True)
    psel2 = jnp.where(col == e0, -2.0, psel)
    p1 = jnp.max(psel2, axis=1, keepdims=True)
    e1 = jnp.min(jnp.where(psel2 >= p1, col, 128.0), axis=1, keepdims=True)
    s = p0 + p1
    g0 = p0 / s
    g1 = p1 / s
    oh0 = (col == e0).astype(jnp.float32)
    oh1 = (col == e1).astype(jnp.float32)
    cnt = oh0 + oh1
    ri = lax.broadcasted_iota(jnp.int32, (T, T), 0)
    ci = lax.broadcasted_iota(jnp.int32, (T, T), 1)
    tri = (ri > ci).astype(jnp.float32)
    c1 = jnp.dot(tri, cnt, preferred_element_type=jnp.float32)
    pos0 = jnp.sum(oh0 * c1, axis=1, keepdims=True)
    pos1 = jnp.sum(oh1 * c1, axis=1, keepdims=True)
    slot0 = e0 * C + pos0
    slot1 = e1 * C + pos1
    v0 = pos0 < C
    v1 = pos1 < C
    sslot0 = jnp.where(v0, slot0, float(EC))
    sslot1 = jnp.where(v1, slot1, float(EC))
    cslot0 = jnp.where(v0, slot0, e0 * C)
    cslot1 = jnp.where(v1, slot1, e1 * C)
    gg0 = jnp.where(v0, g0, 0.0)
    gg1 = jnp.where(v1, g1, 0.0)
    meta = jnp.where(col < 1.0, sslot0,
           jnp.where(col < 2.0, sslot1,
           jnp.where(col < 3.0, cslot0,
           jnp.where(col < 4.0, cslot1,
           jnp.where(col < 5.0, gg0, gg1)))))
    meta_ref[...] = meta


def _router(x, wr_pad):
    return pl.pallas_call(
        _router_body,
        out_shape=jax.ShapeDtypeStruct((T, LANES), jnp.float32),
    )(x, wr_pad)


def _sc_dims():
    try:
        info = plsc.get_sparse_core_info()
        return info.num_cores, info.num_subcores
    except Exception:
        return 2, 16


def _dispatch(x, s0r, s1r):
    NC, NS = _sc_dims()
    NW = NC * NS
    tok_per_w = T // NW
    mesh = plsc.VectorSubcoreMesh(core_axis_name="c", subcore_axis_name="s")

    @functools.partial(
        pl.kernel,
        mesh=mesh,
        out_type=jax.ShapeDtypeStruct((EC + 8, D_MODEL), jnp.float32),
        compiler_params=pltpu.CompilerParams(needs_layout_passes=False),
        scratch_types=[
            pltpu.VMEM((tok_per_w,), jnp.int32),
            pltpu.VMEM((tok_per_w,), jnp.int32),
            pltpu.VMEM((tok_per_w, D_MODEL), jnp.float32),
            pltpu.SemaphoreType.DMA,
            pltpu.SemaphoreType.DMA,
        ],
    )
    def k(x_hbm, s0_hbm, s1_hbm, xb_hbm, i0_v, i1_v, rows_v, sem0, sem1):
        wid = lax.axis_index("s") * NC + lax.axis_index("c")
        tbase = wid * tok_per_w
        pltpu.sync_copy(s0_hbm.at[wid], i0_v)
        pltpu.sync_copy(s1_hbm.at[wid], i1_v)
        pltpu.sync_copy(x_hbm.at[pl.ds(tbase, tok_per_w)], rows_v)
        cp0 = pltpu.async_copy(rows_v, xb_hbm.at[i0_v], sem0)
        cp1 = pltpu.async_copy(rows_v, xb_hbm.at[i1_v], sem1)
        cp0.wait()
        cp1.wait()

    return k(x, s0r, s1r)


def _ffn_body(xb_ref, win_ref, wout_ref, yb_ref):
    f = pl.program_id(1)
    xbf = xb_ref[...].astype(jnp.bfloat16)
    h = jnp.dot(xbf, win_ref[0], preferred_element_type=jnp.float32)
    h = jnp.maximum(h, 0.0).astype(jnp.bfloat16)
    part = jnp.dot(h, wout_ref[0], preferred_element_type=jnp.float32)

    @pl.when(f == 0)
    def _init():
        yb_ref[...] = part

    @pl.when(f > 0)
    def _acc():
        yb_ref[...] = yb_ref[...] + part


def _ffn(xb, w_in, w_out):
    BF = 1024
    nf = D_FF // BF
    return pl.pallas_call(
        _ffn_body,
        grid=(E, nf),
        in_specs=[
            pl.BlockSpec((C, D_MODEL), lambda e, f: (e, 0)),
            pl.BlockSpec((1, D_MODEL, BF), lambda e, f: (e, 0, f)),
            pl.BlockSpec((1, BF, D_MODEL), lambda e, f: (e, f, 0)),
        ],
        out_specs=pl.BlockSpec((C, D_MODEL), lambda e, f: (e, 0)),
        out_shape=jax.ShapeDtypeStruct((EC, D_MODEL), jnp.float32),
    )(xb, w_in.astype(jnp.bfloat16), w_out.astype(jnp.bfloat16))


def _combine(yb, c0, c1, g0, g1):
    NC, NS = _sc_dims()
    NW = NC * NS
    tok_per_w = T // NW
    bt = 16
    n_chunks = tok_per_w // bt
    mesh = plsc.VectorSubcoreMesh(core_axis_name="c", subcore_axis_name="s")

    @functools.partial(
        pl.kernel,
        mesh=mesh,
        out_type=jax.ShapeDtypeStruct((T, D_MODEL), jnp.float32),
        scratch_types=[
            pltpu.VMEM((tok_per_w,), jnp.int32),
            pltpu.VMEM((tok_per_w,), jnp.int32),
            pltpu.VMEM((tok_per_w + 16,), jnp.float32),
            pltpu.VMEM((tok_per_w + 16,), jnp.float32),
            pltpu.VMEM((bt,), jnp.int32),
            pltpu.VMEM((bt,), jnp.int32),
            pltpu.VMEM((bt, D_MODEL), jnp.float32),
            pltpu.VMEM((bt, D_MODEL), jnp.float32),
            pltpu.VMEM((bt, D_MODEL), jnp.float32),
            pltpu.SemaphoreType.DMA,
        ],
    )
    def k(yb_hbm, c0_hbm, c1_hbm, g0_hbm, g1_hbm, out_hbm,
          c0_v, c1_v, g0_v, g1_v, i0_v, i1_v, r0_v, r1_v, o_v, sem):
        wid = lax.axis_index("s") * NC + lax.axis_index("c")
        tbase = wid * tok_per_w
        pltpu.sync_copy(c0_hbm.at[pl.ds(tbase, tok_per_w)], c0_v)
        pltpu.sync_copy(c1_hbm.at[pl.ds(tbase, tok_per_w)], c1_v)
        pltpu.sync_copy(g0_hbm.at[pl.ds(tbase, tok_per_w)], g0_v.at[pl.ds(0, tok_per_w)])
        pltpu.sync_copy(g1_hbm.at[pl.ds(tbase, tok_per_w)], g1_v.at[pl.ds(0, tok_per_w)])

        def chunk_i(ci, carry):
            i0_v[...] = c0_v[pl.ds(ci * bt, bt)]
            i1_v[...] = c1_v[pl.ds(ci * bt, bt)]
            pltpu.async_copy(yb_hbm.at[i0_v], r0_v, sem).wait()
            pltpu.async_copy(yb_hbm.at[i1_v], r1_v, sem).wait()

            def row_i(r, c2):
                a = g0_v[pl.ds(ci * bt + r, 16)][0]
                b = g1_v[pl.ds(ci * bt + r, 16)][0]

                def lane_i(j, c3):
                    o_v[r, pl.ds(j * 16, 16)] = (
                        a * r0_v[r, pl.ds(j * 16, 16)]
                        + b * r1_v[r, pl.ds(j * 16, 16)])
                    return c3
                lax.fori_loop(0, D_MODEL // 16, lane_i, 0)
                return c2
            lax.fori_loop(0, bt, row_i, 0)
            pltpu.sync_copy(o_v, out_hbm.at[pl.ds(tbase + ci * bt, bt)])
            return carry
        lax.fori_loop(0, n_chunks, chunk_i, 0)

    return k(yb, c0, c1, g0, g1)


def kernel(x, w_router, w_in, w_out):
    NC, NS = _sc_dims()
    NW = NC * NS
    wr_pad = jnp.zeros((D_MODEL, LANES), jnp.float32).at[:, :E].set(w_router)
    meta = _router(x, wr_pad)
    s0 = meta[:, 0].astype(jnp.int32).reshape(NW, T // NW)
    s1 = meta[:, 1].astype(jnp.int32).reshape(NW, T // NW)
    c0 = meta[:, 2].astype(jnp.int32)
    c1 = meta[:, 3].astype(jnp.int32)
    g0 = meta[:, 4]
    g1 = meta[:, 5]
    xb = _dispatch(x, s0, s1)
    yb = _ffn(xb, w_in, w_out)
    return _combine(yb, c0, c1, g0, g1)


# speedup vs baseline: 1.4782x; 1.4782x over previous
import functools

import jax
import jax.numpy as jnp
from jax import lax
from jax.experimental import pallas as pl
from jax.experimental.pallas import tpu as pltpu
from jax.experimental.pallas import tpu_sc as plsc

E = 8
TOP_K = 2
D_MODEL = 1024
D_FF = 4096
T = 2048
C = 1024
EC = E * C
LANES = 128


def _router_body(x_ref, wr_ref, meta_ref):
    x = x_ref[...]
    wr = wr_ref[...]
    logits = jnp.dot(x, wr, preferred_element_type=jnp.float32)
    col = lax.broadcasted_iota(jnp.int32, (T, LANES), 1).astype(jnp.float32)
    real = col < E
    logits = jnp.where(real, logits, -1e30)
    mx = jnp.max(logits, axis=1, keepdims=True)
    ex = jnp.exp(logits - mx)
    probs = ex / jnp.sum(ex, axis=1, keepdims=True)
    psel = jnp.where(real, probs, -1.0)
    p0 = jnp.max(psel, axis=1, keepdims=True)
    e0 = jnp.min(jnp.where(psel >= p0, col, 128.0), axis=1, keepdims=True)
    psel2 = jnp.where(col == e0, -2.0, psel)
    p1 = jnp.max(psel2, axis=1, keepdims=True)
    e1 = jnp.min(jnp.where(psel2 >= p1, col, 128.0), axis=1, keepdims=True)
    s = p0 + p1
    g0 = p0 / s
    g1 = p1 / s
    oh0 = (col == e0).astype(jnp.float32)
    oh1 = (col == e1).astype(jnp.float32)
    cnt = oh0 + oh1
    ri = lax.broadcasted_iota(jnp.int32, (T, T), 0)
    ci = lax.broadcasted_iota(jnp.int32, (T, T), 1)
    tri = (ri > ci).astype(jnp.float32)
    c1 = jnp.dot(tri, cnt, preferred_element_type=jnp.float32)
    pos0 = jnp.sum(oh0 * c1, axis=1, keepdims=True)
    pos1 = jnp.sum(oh1 * c1, axis=1, keepdims=True)
    slot0 = e0 * C + pos0
    slot1 = e1 * C + pos1
    v0 = pos0 < C
    v1 = pos1 < C
    sslot0 = jnp.where(v0, slot0, float(EC))
    sslot1 = jnp.where(v1, slot1, float(EC))
    cslot0 = jnp.where(v0, slot0, e0 * C)
    cslot1 = jnp.where(v1, slot1, e1 * C)
    gg0 = jnp.where(v0, g0, 0.0)
    gg1 = jnp.where(v1, g1, 0.0)
    meta = jnp.where(col < 1.0, sslot0,
           jnp.where(col < 2.0, sslot1,
           jnp.where(col < 3.0, cslot0,
           jnp.where(col < 4.0, cslot1,
           jnp.where(col < 5.0, gg0, gg1)))))
    meta_ref[...] = meta


def _router(x, wr_pad):
    return pl.pallas_call(
        _router_body,
        out_shape=jax.ShapeDtypeStruct((T, LANES), jnp.float32),
    )(x, wr_pad)


def _sc_dims():
    try:
        info = plsc.get_sparse_core_info()
        return info.num_cores, info.num_subcores
    except Exception:
        return 2, 16


def _dispatch(x, s0r, s1r):
    NC, NS = _sc_dims()
    NW = NC * NS
    tok_per_w = T // NW
    mesh = plsc.VectorSubcoreMesh(core_axis_name="c", subcore_axis_name="s")

    @functools.partial(
        pl.kernel,
        mesh=mesh,
        out_type=jax.ShapeDtypeStruct((EC + 8, D_MODEL), jnp.float32),
        compiler_params=pltpu.CompilerParams(needs_layout_passes=False),
        scratch_types=[
            pltpu.VMEM((tok_per_w,), jnp.int32),
            pltpu.VMEM((tok_per_w,), jnp.int32),
            pltpu.VMEM((tok_per_w, D_MODEL), jnp.float32),
            pltpu.SemaphoreType.DMA,
            pltpu.SemaphoreType.DMA,
        ],
    )
    def k(x_hbm, s0_hbm, s1_hbm, xb_hbm, i0_v, i1_v, rows_v, sem0, sem1):
        wid = lax.axis_index("s") * NC + lax.axis_index("c")
        tbase = wid * tok_per_w
        pltpu.sync_copy(s0_hbm.at[wid], i0_v)
        pltpu.sync_copy(s1_hbm.at[wid], i1_v)
        pltpu.sync_copy(x_hbm.at[pl.ds(tbase, tok_per_w)], rows_v)
        cp0 = pltpu.async_copy(rows_v, xb_hbm.at[i0_v], sem0)
        cp1 = pltpu.async_copy(rows_v, xb_hbm.at[i1_v], sem1)
        cp0.wait()
        cp1.wait()

    return k(x, s0r, s1r)


def _ffn_body(xb_ref, win_ref, wout_ref, yb_ref):
    f = pl.program_id(1)
    h = jnp.dot(xb_ref[...], win_ref[0], preferred_element_type=jnp.float32)
    h = jnp.maximum(h, 0.0)
    part = jnp.dot(h, wout_ref[0], preferred_element_type=jnp.float32)

    @pl.when(f == 0)
    def _init():
        yb_ref[...] = part

    @pl.when(f > 0)
    def _acc():
        yb_ref[...] = yb_ref[...] + part


def _ffn(xb, w_in, w_out):
    BF = 1024
    nf = D_FF // BF
    return pl.pallas_call(
        _ffn_body,
        grid=(E, nf),
        in_specs=[
            pl.BlockSpec((C, D_MODEL), lambda e, f: (e, 0)),
            pl.BlockSpec((1, D_MODEL, BF), lambda e, f: (e, 0, f)),
            pl.BlockSpec((1, BF, D_MODEL), lambda e, f: (e, f, 0)),
        ],
        out_specs=pl.BlockSpec((C, D_MODEL), lambda e, f: (e, 0)),
        out_shape=jax.ShapeDtypeStruct((EC, D_MODEL), jnp.float32),
    )(xb, w_in, w_out)


def _combine(yb, c0, c1, g0, g1):
    NC, NS = _sc_dims()
    NW = NC * NS
    tok_per_w = T // NW
    bt = 16
    n_chunks = tok_per_w // bt
    mesh = plsc.VectorSubcoreMesh(core_axis_name="c", subcore_axis_name="s")

    @functools.partial(
        pl.kernel,
        mesh=mesh,
        out_type=jax.ShapeDtypeStruct((T, D_MODEL), jnp.float32),
        scratch_types=[
            pltpu.VMEM((tok_per_w,), jnp.int32),
            pltpu.VMEM((tok_per_w,), jnp.int32),
            pltpu.VMEM((tok_per_w + 16,), jnp.float32),
            pltpu.VMEM((tok_per_w + 16,), jnp.float32),
            pltpu.VMEM((bt,), jnp.int32),
            pltpu.VMEM((bt,), jnp.int32),
            pltpu.VMEM((bt, D_MODEL), jnp.float32),
            pltpu.VMEM((bt, D_MODEL), jnp.float32),
            pltpu.VMEM((bt, D_MODEL), jnp.float32),
            pltpu.SemaphoreType.DMA,
        ],
    )
    def k(yb_hbm, c0_hbm, c1_hbm, g0_hbm, g1_hbm, out_hbm,
          c0_v, c1_v, g0_v, g1_v, i0_v, i1_v, r0_v, r1_v, o_v, sem):
        wid = lax.axis_index("s") * NC + lax.axis_index("c")
        tbase = wid * tok_per_w
        pltpu.sync_copy(c0_hbm.at[pl.ds(tbase, tok_per_w)], c0_v)
        pltpu.sync_copy(c1_hbm.at[pl.ds(tbase, tok_per_w)], c1_v)
        pltpu.sync_copy(g0_hbm.at[pl.ds(tbase, tok_per_w)], g0_v.at[pl.ds(0, tok_per_w)])
        pltpu.sync_copy(g1_hbm.at[pl.ds(tbase, tok_per_w)], g1_v.at[pl.ds(0, tok_per_w)])

        def chunk_i(ci, carry):
            i0_v[...] = c0_v[pl.ds(ci * bt, bt)]
            i1_v[...] = c1_v[pl.ds(ci * bt, bt)]
            pltpu.async_copy(yb_hbm.at[i0_v], r0_v, sem).wait()
            pltpu.async_copy(yb_hbm.at[i1_v], r1_v, sem).wait()

            def row_i(r, c2):
                a = g0_v[pl.ds(ci * bt + r, 16)][0]
                b = g1_v[pl.ds(ci * bt + r, 16)][0]

                def lane_i(j, c3):
                    o_v[r, pl.ds(j * 16, 16)] = (
                        a * r0_v[r, pl.ds(j * 16, 16)]
                        + b * r1_v[r, pl.ds(j * 16, 16)])
                    return c3
                lax.fori_loop(0, D_MODEL // 16, lane_i, 0)
                return c2
            lax.fori_loop(0, bt, row_i, 0)
            pltpu.sync_copy(o_v, out_hbm.at[pl.ds(tbase + ci * bt, bt)])
            return carry
        lax.fori_loop(0, n_chunks, chunk_i, 0)

    return k(yb, c0, c1, g0, g1)


def kernel(x, w_router, w_in, w_out):
    NC, NS = _sc_dims()
    NW = NC * NS
    wr_pad = jnp.zeros((D_MODEL, LANES), jnp.float32).at[:, :E].set(w_router)
    meta = _router(x, wr_pad)
    s0 = meta[:, 0].astype(jnp.int32).reshape(NW, T // NW)
    s1 = meta[:, 1].astype(jnp.int32).reshape(NW, T // NW)
    c0 = meta[:, 2].astype(jnp.int32)
    c1 = meta[:, 3].astype(jnp.int32)
    g0 = meta[:, 4]
    g1 = meta[:, 5]
    xb = _dispatch(x, s0, s1)
    yb = _ffn(xb, w_in, w_out)
    return _combine(yb, c0, c1, g0, g1)


# speedup vs baseline: 1.8145x; 1.2275x over previous
import functools

import jax
import jax.numpy as jnp
from jax import lax
from jax.experimental import pallas as pl
from jax.experimental.pallas import tpu as pltpu
from jax.experimental.pallas import tpu_sc as plsc

E = 8
TOP_K = 2
D_MODEL = 1024
D_FF = 4096
T = 2048
C = 1024
EC = E * C
LANES = 128


def _router_body(x_ref, wr_ref, meta_ref, nb_ref):
    x = x_ref[...]
    wr = wr_ref[...]
    logits = jnp.dot(x, wr, preferred_element_type=jnp.float32)
    col = lax.broadcasted_iota(jnp.int32, (T, LANES), 1).astype(jnp.float32)
    real = col < E
    logits = jnp.where(real, logits, -1e30)
    mx = jnp.max(logits, axis=1, keepdims=True)
    ex = jnp.exp(logits - mx)
    probs = ex / jnp.sum(ex, axis=1, keepdims=True)
    psel = jnp.where(real, probs, -1.0)
    p0 = jnp.max(psel, axis=1, keepdims=True)
    e0 = jnp.min(jnp.where(psel >= p0, col, 128.0), axis=1, keepdims=True)
    psel2 = jnp.where(col == e0, -2.0, psel)
    p1 = jnp.max(psel2, axis=1, keepdims=True)
    e1 = jnp.min(jnp.where(psel2 >= p1, col, 128.0), axis=1, keepdims=True)
    s = p0 + p1
    g0 = p0 / s
    g1 = p1 / s
    oh0 = (col == e0).astype(jnp.float32)
    oh1 = (col == e1).astype(jnp.float32)
    cnt = oh0 + oh1
    ri = lax.broadcasted_iota(jnp.int32, (T, T), 0)
    ci = lax.broadcasted_iota(jnp.int32, (T, T), 1)
    tri = (ri > ci).astype(jnp.float32)
    c1 = jnp.dot(tri, cnt, preferred_element_type=jnp.float32)
    pos0 = jnp.sum(oh0 * c1, axis=1, keepdims=True)
    pos1 = jnp.sum(oh1 * c1, axis=1, keepdims=True)
    slot0 = e0 * C + pos0
    slot1 = e1 * C + pos1
    v0 = pos0 < C
    v1 = pos1 < C
    sslot0 = jnp.where(v0, slot0, float(EC))
    sslot1 = jnp.where(v1, slot1, float(EC))
    cslot0 = jnp.where(v0, slot0, e0 * C)
    cslot1 = jnp.where(v1, slot1, e1 * C)
    gg0 = jnp.where(v0, g0, 0.0)
    gg1 = jnp.where(v1, g1, 0.0)
    meta = jnp.where(col < 1.0, sslot0,
           jnp.where(col < 2.0, sslot1,
           jnp.where(col < 3.0, cslot0,
           jnp.where(col < 4.0, cslot1,
           jnp.where(col < 5.0, gg0, gg1)))))
    meta_ref[...] = meta
    sz = jnp.minimum(jnp.sum(cnt, axis=0, keepdims=True), float(C))
    nb_ref[...] = jnp.floor((sz + 255.0) * (1.0 / 256.0))


def _router(x, wr_pad):
    return pl.pallas_call(
        _router_body,
        out_shape=[
            jax.ShapeDtypeStruct((T, LANES), jnp.float32),
            jax.ShapeDtypeStruct((1, LANES), jnp.float32),
        ],
    )(x, wr_pad)


def _sc_dims():
    try:
        info = plsc.get_sparse_core_info()
        return info.num_cores, info.num_subcores
    except Exception:
        return 2, 16


def _dispatch(x, s0r, s1r):
    NC, NS = _sc_dims()
    NW = NC * NS
    tok_per_w = T // NW
    mesh = plsc.VectorSubcoreMesh(core_axis_name="c", subcore_axis_name="s")

    @functools.partial(
        pl.kernel,
        mesh=mesh,
        out_type=jax.ShapeDtypeStruct((EC + 8, D_MODEL), jnp.float32),
        compiler_params=pltpu.CompilerParams(needs_layout_passes=False),
        scratch_types=[
            pltpu.VMEM((tok_per_w,), jnp.int32),
            pltpu.VMEM((tok_per_w,), jnp.int32),
            pltpu.VMEM((tok_per_w, D_MODEL), jnp.float32),
            pltpu.SemaphoreType.DMA,
            pltpu.SemaphoreType.DMA,
        ],
    )
    def k(x_hbm, s0_hbm, s1_hbm, xb_hbm, i0_v, i1_v, rows_v, sem0, sem1):
        wid = lax.axis_index("s") * NC + lax.axis_index("c")
        tbase = wid * tok_per_w
        pltpu.sync_copy(s0_hbm.at[wid], i0_v)
        pltpu.sync_copy(s1_hbm.at[wid], i1_v)
        pltpu.sync_copy(x_hbm.at[pl.ds(tbase, tok_per_w)], rows_v)
        cp0 = pltpu.async_copy(rows_v, xb_hbm.at[i0_v], sem0)
        cp1 = pltpu.async_copy(rows_v, xb_hbm.at[i1_v], sem1)
        cp0.wait()
        cp1.wait()

    return k(x, s0r, s1r)


_SUB = 256


def _ffn_body(nb_ref, xb_ref, win_ref, wout_ref, yb_ref):
    e = pl.program_id(0)
    f = pl.program_id(1)
    n = nb_ref[0, e]
    for i in range(C // _SUB):
        @pl.when(i < n)
        def _sub():
            xs = xb_ref[pl.ds(i * _SUB, _SUB), :]
            h = jnp.dot(xs, win_ref[0], preferred_element_type=jnp.float32)
            h = jnp.maximum(h, 0.0)
            part = jnp.dot(h, wout_ref[0], preferred_element_type=jnp.float32)

            @pl.when(f == 0)
            def _init():
                yb_ref[pl.ds(i * _SUB, _SUB), :] = part

            @pl.when(f > 0)
            def _acc():
                yb_ref[pl.ds(i * _SUB, _SUB), :] = (
                    yb_ref[pl.ds(i * _SUB, _SUB), :] + part)


def _ffn(nb, xb, w_in, w_out):
    BF = 1024
    nf = D_FF // BF
    return pl.pallas_call(
        _ffn_body,
        grid=(E, nf),
        in_specs=[
            pl.BlockSpec(memory_space=pltpu.SMEM),
            pl.BlockSpec((C, D_MODEL), lambda e, f: (e, 0)),
            pl.BlockSpec((1, D_MODEL, BF), lambda e, f: (e, 0, f)),
            pl.BlockSpec((1, BF, D_MODEL), lambda e, f: (e, f, 0)),
        ],
        out_specs=pl.BlockSpec((C, D_MODEL), lambda e, f: (e, 0)),
        out_shape=jax.ShapeDtypeStruct((EC, D_MODEL), jnp.float32),
    )(nb, xb, w_in, w_out)


def _combine(yb, c0, c1, g0, g1):
    NC, NS = _sc_dims()
    NW = NC * NS
    tok_per_w = T // NW
    bt = 16
    n_chunks = tok_per_w // bt
    mesh = plsc.VectorSubcoreMesh(core_axis_name="c", subcore_axis_name="s")

    @functools.partial(
        pl.kernel,
        mesh=mesh,
        out_type=jax.ShapeDtypeStruct((T, D_MODEL), jnp.float32),
        scratch_types=[
            pltpu.VMEM((tok_per_w,), jnp.int32),
            pltpu.VMEM((tok_per_w,), jnp.int32),
            pltpu.VMEM((tok_per_w + 16,), jnp.float32),
            pltpu.VMEM((tok_per_w + 16,), jnp.float32),
            pltpu.VMEM((bt,), jnp.int32),
            pltpu.VMEM((bt,), jnp.int32),
            pltpu.VMEM((bt, D_MODEL), jnp.float32),
            pltpu.VMEM((bt, D_MODEL), jnp.float32),
            pltpu.VMEM((bt, D_MODEL), jnp.float32),
            pltpu.SemaphoreType.DMA,
        ],
    )
    def k(yb_hbm, c0_hbm, c1_hbm, g0_hbm, g1_hbm, out_hbm,
          c0_v, c1_v, g0_v, g1_v, i0_v, i1_v, r0_v, r1_v, o_v, sem):
        wid = lax.axis_index("s") * NC + lax.axis_index("c")
        tbase = wid * tok_per_w
        pltpu.sync_copy(c0_hbm.at[pl.ds(tbase, tok_per_w)], c0_v)
        pltpu.sync_copy(c1_hbm.at[pl.ds(tbase, tok_per_w)], c1_v)
        pltpu.sync_copy(g0_hbm.at[pl.ds(tbase, tok_per_w)], g0_v.at[pl.ds(0, tok_per_w)])
        pltpu.sync_copy(g1_hbm.at[pl.ds(tbase, tok_per_w)], g1_v.at[pl.ds(0, tok_per_w)])

        def chunk_i(ci, carry):
            i0_v[...] = c0_v[pl.ds(ci * bt, bt)]
            i1_v[...] = c1_v[pl.ds(ci * bt, bt)]
            cp0 = pltpu.async_copy(yb_hbm.at[i0_v], r0_v, sem)
            cp1 = pltpu.async_copy(yb_hbm.at[i1_v], r1_v, sem)
            cp0.wait()
            cp1.wait()

            def row_i(r, c2):
                a = g0_v[pl.ds(ci * bt + r, 16)][0]
                b = g1_v[pl.ds(ci * bt + r, 16)][0]

                def lane_i(j, c3):
                    for u in range(4):
                        sl = pl.ds(j * 64 + u * 16, 16)
                        o_v[r, sl] = a * r0_v[r, sl] + b * r1_v[r, sl]
                    return c3
                lax.fori_loop(0, D_MODEL // 64, lane_i, 0)
                return c2
            lax.fori_loop(0, bt, row_i, 0)
            pltpu.sync_copy(o_v, out_hbm.at[pl.ds(tbase + ci * bt, bt)])
            return carry
        lax.fori_loop(0, n_chunks, chunk_i, 0)

    return k(yb, c0, c1, g0, g1)


def kernel(x, w_router, w_in, w_out):
    NC, NS = _sc_dims()
    NW = NC * NS
    wr_pad = jnp.zeros((D_MODEL, LANES), jnp.float32).at[:, :E].set(w_router)
    meta, nb_row = _router(x, wr_pad)
    nb = nb_row[:, :E].astype(jnp.int32)
    s0 = meta[:, 0].astype(jnp.int32).reshape(NW, T // NW)
    s1 = meta[:, 1].astype(jnp.int32).reshape(NW, T // NW)
    c0 = meta[:, 2].astype(jnp.int32)
    c1 = meta[:, 3].astype(jnp.int32)
    g0 = meta[:, 4]
    g1 = meta[:, 5]
    xb = _dispatch(x, s0, s1)
    yb = _ffn(nb, xb, w_in, w_out)
    return _combine(yb, c0, c1, g0, g1)


# speedup vs baseline: 1.9572x; 1.0787x over previous
import functools

import jax
import jax.numpy as jnp
from jax import lax
from jax.experimental import pallas as pl
from jax.experimental.pallas import tpu as pltpu
from jax.experimental.pallas import tpu_sc as plsc

E = 8
TOP_K = 2
D_MODEL = 1024
D_FF = 4096
T = 2048
C = 1024
EC = E * C
LANES = 128


def _router_body(x_ref, wr_ref, meta_ref, nb_ref):
    x = x_ref[...]
    wr = wr_ref[...]
    logits = jnp.dot(x, wr, preferred_element_type=jnp.float32)
    col = lax.broadcasted_iota(jnp.int32, (T, LANES), 1).astype(jnp.float32)
    real = col < E
    logits = jnp.where(real, logits, -1e30)
    mx = jnp.max(logits, axis=1, keepdims=True)
    ex = jnp.exp(logits - mx)
    probs = ex / jnp.sum(ex, axis=1, keepdims=True)
    psel = jnp.where(real, probs, -1.0)
    p0 = jnp.max(psel, axis=1, keepdims=True)
    e0 = jnp.min(jnp.where(psel >= p0, col, 128.0), axis=1, keepdims=True)
    psel2 = jnp.where(col == e0, -2.0, psel)
    p1 = jnp.max(psel2, axis=1, keepdims=True)
    e1 = jnp.min(jnp.where(psel2 >= p1, col, 128.0), axis=1, keepdims=True)
    s = p0 + p1
    g0 = p0 / s
    g1 = p1 / s
    oh0 = (col == e0).astype(jnp.float32)
    oh1 = (col == e1).astype(jnp.float32)
    cnt = oh0 + oh1
    ri = lax.broadcasted_iota(jnp.int32, (T, T), 0)
    ci = lax.broadcasted_iota(jnp.int32, (T, T), 1)
    tri = (ri > ci).astype(jnp.float32)
    c1 = jnp.dot(tri, cnt, preferred_element_type=jnp.float32)
    pos0 = jnp.sum(oh0 * c1, axis=1, keepdims=True)
    pos1 = jnp.sum(oh1 * c1, axis=1, keepdims=True)
    slot0 = e0 * C + pos0
    slot1 = e1 * C + pos1
    v0 = pos0 < C
    v1 = pos1 < C
    sslot0 = jnp.where(v0, slot0, float(EC))
    sslot1 = jnp.where(v1, slot1, float(EC))
    cslot0 = jnp.where(v0, slot0, e0 * C)
    cslot1 = jnp.where(v1, slot1, e1 * C)
    gg0 = jnp.where(v0, g0, 0.0)
    gg1 = jnp.where(v1, g1, 0.0)
    meta = jnp.where(col < 1.0, sslot0,
           jnp.where(col < 2.0, sslot1,
           jnp.where(col < 3.0, cslot0,
           jnp.where(col < 4.0, cslot1,
           jnp.where(col < 5.0, gg0, gg1)))))
    meta_ref[...] = meta
    sz = jnp.minimum(jnp.sum(cnt, axis=0, keepdims=True), float(C))
    nb_ref[...] = jnp.floor((sz + 255.0) * (1.0 / 256.0))


def _router(x, wr_pad):
    return pl.pallas_call(
        _router_body,
        out_shape=[
            jax.ShapeDtypeStruct((T, LANES), jnp.float32),
            jax.ShapeDtypeStruct((1, LANES), jnp.float32),
        ],
    )(x, wr_pad)


def _sc_dims():
    try:
        info = plsc.get_sparse_core_info()
        return info.num_cores, info.num_subcores
    except Exception:
        return 2, 16


def _dispatch(x, s0r, s1r):
    NC, NS = _sc_dims()
    NW = NC * NS
    tok_per_w = T // NW
    mesh = plsc.VectorSubcoreMesh(core_axis_name="c", subcore_axis_name="s")

    @functools.partial(
        pl.kernel,
        mesh=mesh,
        out_type=jax.ShapeDtypeStruct((EC + 8, D_MODEL), jnp.float32),
        compiler_params=pltpu.CompilerParams(needs_layout_passes=False),
        scratch_types=[
            pltpu.VMEM((tok_per_w,), jnp.int32),
            pltpu.VMEM((tok_per_w,), jnp.int32),
            pltpu.VMEM((tok_per_w, D_MODEL), jnp.float32),
            pltpu.SemaphoreType.DMA,
            pltpu.SemaphoreType.DMA,
        ],
    )
    def k(x_hbm, s0_hbm, s1_hbm, xb_hbm, i0_v, i1_v, rows_v, sem0, sem1):
        wid = lax.axis_index("s") * NC + lax.axis_index("c")
        tbase = wid * tok_per_w
        pltpu.sync_copy(s0_hbm.at[wid], i0_v)
        pltpu.sync_copy(s1_hbm.at[wid], i1_v)
        pltpu.sync_copy(x_hbm.at[pl.ds(tbase, tok_per_w)], rows_v)
        cp0 = pltpu.async_copy(rows_v, xb_hbm.at[i0_v], sem0)
        cp1 = pltpu.async_copy(rows_v, xb_hbm.at[i1_v], sem1)
        cp0.wait()
        cp1.wait()

    return k(x, s0r, s1r)


_SUB = 256


def _ffn_body(nb_ref, xb_ref, win_ref, wout_ref, yb_ref):
    e = pl.program_id(0)
    f = pl.program_id(1)
    n = nb_ref[0, e]
    for i in range(C // _SUB):
        @pl.when(i < n)
        def _sub():
            xs = xb_ref[pl.ds(i * _SUB, _SUB), :]
            h = jnp.dot(xs, win_ref[0], preferred_element_type=jnp.float32)
            h = jnp.maximum(h, 0.0)
            part = jnp.dot(h, wout_ref[0], preferred_element_type=jnp.float32)

            @pl.when(f == 0)
            def _init():
                yb_ref[pl.ds(i * _SUB, _SUB), :] = part

            @pl.when(f > 0)
            def _acc():
                yb_ref[pl.ds(i * _SUB, _SUB), :] = (
                    yb_ref[pl.ds(i * _SUB, _SUB), :] + part)


def _ffn(nb, xb, w_in, w_out):
    BF = 2048
    nf = D_FF // BF
    return pl.pallas_call(
        _ffn_body,
        grid=(E, nf),
        in_specs=[
            pl.BlockSpec(memory_space=pltpu.SMEM),
            pl.BlockSpec((C, D_MODEL), lambda e, f: (e, 0)),
            pl.BlockSpec((1, D_MODEL, BF), lambda e, f: (e, 0, f)),
            pl.BlockSpec((1, BF, D_MODEL), lambda e, f: (e, f, 0)),
        ],
        out_specs=pl.BlockSpec((C, D_MODEL), lambda e, f: (e, 0)),
        out_shape=jax.ShapeDtypeStruct((EC, D_MODEL), jnp.float32),
    )(nb, xb, w_in, w_out)


def _combine(yb, c0, c1, g0, g1):
    NC, NS = _sc_dims()
    NW = NC * NS
    tok_per_w = T // NW
    bt = 16
    n_chunks = tok_per_w // bt
    mesh = plsc.VectorSubcoreMesh(core_axis_name="c", subcore_axis_name="s")

    @functools.partial(
        pl.kernel,
        mesh=mesh,
        out_type=jax.ShapeDtypeStruct((T, D_MODEL), jnp.float32),
        scratch_types=[
            pltpu.VMEM((tok_per_w,), jnp.int32),
            pltpu.VMEM((tok_per_w,), jnp.int32),
            pltpu.VMEM((tok_per_w + 16,), jnp.float32),
            pltpu.VMEM((tok_per_w + 16,), jnp.float32),
            pltpu.VMEM((bt,), jnp.int32),
            pltpu.VMEM((bt,), jnp.int32),
            pltpu.VMEM((bt, D_MODEL), jnp.float32),
            pltpu.VMEM((bt, D_MODEL), jnp.float32),
            pltpu.VMEM((bt, D_MODEL), jnp.float32),
            pltpu.SemaphoreType.DMA,
        ],
    )
    def k(yb_hbm, c0_hbm, c1_hbm, g0_hbm, g1_hbm, out_hbm,
          c0_v, c1_v, g0_v, g1_v, i0_v, i1_v, r0_v, r1_v, o_v, sem):
        wid = lax.axis_index("s") * NC + lax.axis_index("c")
        tbase = wid * tok_per_w
        pltpu.sync_copy(c0_hbm.at[pl.ds(tbase, tok_per_w)], c0_v)
        pltpu.sync_copy(c1_hbm.at[pl.ds(tbase, tok_per_w)], c1_v)
        pltpu.sync_copy(g0_hbm.at[pl.ds(tbase, tok_per_w)], g0_v.at[pl.ds(0, tok_per_w)])
        pltpu.sync_copy(g1_hbm.at[pl.ds(tbase, tok_per_w)], g1_v.at[pl.ds(0, tok_per_w)])

        def chunk_i(ci, carry):
            i0_v[...] = c0_v[pl.ds(ci * bt, bt)]
            i1_v[...] = c1_v[pl.ds(ci * bt, bt)]
            cp0 = pltpu.async_copy(yb_hbm.at[i0_v], r0_v, sem)
            cp1 = pltpu.async_copy(yb_hbm.at[i1_v], r1_v, sem)
            cp0.wait()
            cp1.wait()

            def row_i(r, c2):
                a = g0_v[pl.ds(ci * bt + r, 16)][0]
                b = g1_v[pl.ds(ci * bt + r, 16)][0]

                def lane_i(j, c3):
                    for u in range(4):
                        sl = pl.ds(j * 64 + u * 16, 16)
                        o_v[r, sl] = a * r0_v[r, sl] + b * r1_v[r, sl]
                    return c3
                lax.fori_loop(0, D_MODEL // 64, lane_i, 0)
                return c2
            lax.fori_loop(0, bt, row_i, 0)
            pltpu.sync_copy(o_v, out_hbm.at[pl.ds(tbase + ci * bt, bt)])
            return carry
        lax.fori_loop(0, n_chunks, chunk_i, 0)

    return k(yb, c0, c1, g0, g1)


def kernel(x, w_router, w_in, w_out):
    NC, NS = _sc_dims()
    NW = NC * NS
    wr_pad = jnp.zeros((D_MODEL, LANES), jnp.float32).at[:, :E].set(w_router)
    meta, nb_row = _router(x, wr_pad)
    nb = nb_row[:, :E].astype(jnp.int32)
    s0 = meta[:, 0].astype(jnp.int32).reshape(NW, T // NW)
    s1 = meta[:, 1].astype(jnp.int32).reshape(NW, T // NW)
    c0 = meta[:, 2].astype(jnp.int32)
    c1 = meta[:, 3].astype(jnp.int32)
    g0 = meta[:, 4]
    g1 = meta[:, 5]
    xb = _dispatch(x, s0, s1)
    yb = _ffn(nb, xb, w_in, w_out)
    return _combine(yb, c0, c1, g0, g1)


# speedup vs baseline: 2.1009x; 1.0734x over previous
import functools

import jax
import jax.numpy as jnp
from jax import lax
from jax.experimental import pallas as pl
from jax.experimental.pallas import tpu as pltpu
from jax.experimental.pallas import tpu_sc as plsc

E = 8
TOP_K = 2
D_MODEL = 1024
D_FF = 4096
T = 2048
C = 1024
EC = E * C
LANES = 128


def _router_body(x_ref, wr_ref, meta_ref, nb_ref):
    x = x_ref[...]
    wr = wr_ref[...]
    logits = jnp.dot(x, wr, preferred_element_type=jnp.float32)
    col = lax.broadcasted_iota(jnp.int32, (T, LANES), 1).astype(jnp.float32)
    real = col < E
    logits = jnp.where(real, logits, -1e30)
    mx = jnp.max(logits, axis=1, keepdims=True)
    ex = jnp.exp(logits - mx)
    probs = ex / jnp.sum(ex, axis=1, keepdims=True)
    psel = jnp.where(real, probs, -1.0)
    p0 = jnp.max(psel, axis=1, keepdims=True)
    e0 = jnp.min(jnp.where(psel >= p0, col, 128.0), axis=1, keepdims=True)
    psel2 = jnp.where(col == e0, -2.0, psel)
    p1 = jnp.max(psel2, axis=1, keepdims=True)
    e1 = jnp.min(jnp.where(psel2 >= p1, col, 128.0), axis=1, keepdims=True)
    s = p0 + p1
    g0 = p0 / s
    g1 = p1 / s
    oh0 = (col == e0).astype(jnp.float32)
    oh1 = (col == e1).astype(jnp.float32)
    cnt = oh0 + oh1
    ri = lax.broadcasted_iota(jnp.int32, (T, T), 0)
    ci = lax.broadcasted_iota(jnp.int32, (T, T), 1)
    tri = (ri > ci).astype(jnp.float32)
    c1 = jnp.dot(tri, cnt, preferred_element_type=jnp.float32)
    pos0 = jnp.sum(oh0 * c1, axis=1, keepdims=True)
    pos1 = jnp.sum(oh1 * c1, axis=1, keepdims=True)
    slot0 = e0 * C + pos0
    slot1 = e1 * C + pos1
    v0 = pos0 < C
    v1 = pos1 < C
    sslot0 = jnp.where(v0, slot0, float(EC))
    sslot1 = jnp.where(v1, slot1, float(EC))
    cslot0 = jnp.where(v0, slot0, e0 * C)
    cslot1 = jnp.where(v1, slot1, e1 * C)
    gg0 = jnp.where(v0, g0, 0.0)
    gg1 = jnp.where(v1, g1, 0.0)
    meta = jnp.where(col < 1.0, sslot0,
           jnp.where(col < 2.0, sslot1,
           jnp.where(col < 3.0, cslot0,
           jnp.where(col < 4.0, cslot1,
           jnp.where(col < 5.0, gg0, gg1)))))
    meta_ref[...] = meta
    sz = jnp.minimum(jnp.sum(cnt, axis=0, keepdims=True), float(C))
    nb_ref[...] = jnp.floor((sz + 255.0) * (1.0 / 256.0))


def _router(x, wr_pad):
    return pl.pallas_call(
        _router_body,
        out_shape=[
            jax.ShapeDtypeStruct((T, LANES), jnp.float32),
            jax.ShapeDtypeStruct((1, LANES), jnp.float32),
        ],
    )(x, wr_pad)


def _sc_dims():
    try:
        info = plsc.get_sparse_core_info()
        return info.num_cores, info.num_subcores
    except Exception:
        return 2, 16


def _dispatch(x, meta):
    NC, NS = _sc_dims()
    NW = NC * NS
    tok_per_w = T // NW
    mesh = plsc.VectorSubcoreMesh(core_axis_name="c", subcore_axis_name="s")

    @functools.partial(
        pl.kernel,
        mesh=mesh,
        out_type=jax.ShapeDtypeStruct((EC + 8, D_MODEL), jnp.float32),
        compiler_params=pltpu.CompilerParams(needs_layout_passes=False),
        scratch_types=[
            pltpu.VMEM((tok_per_w, LANES), jnp.float32),
            pltpu.VMEM((tok_per_w,), jnp.int32),
            pltpu.VMEM((tok_per_w,), jnp.int32),
            pltpu.VMEM((tok_per_w, D_MODEL), jnp.float32),
            pltpu.SemaphoreType.DMA,
            pltpu.SemaphoreType.DMA,
        ],
    )
    def k(x_hbm, meta_hbm, xb_hbm, m_v, i0_v, i1_v, rows_v, sem0, sem1):
        wid = lax.axis_index("s") * NC + lax.axis_index("c")
        tbase = wid * tok_per_w
        cpm = pltpu.async_copy(meta_hbm.at[pl.ds(tbase, tok_per_w)], m_v, sem0)
        cpx = pltpu.async_copy(x_hbm.at[pl.ds(tbase, tok_per_w)], rows_v, sem1)
        cpm.wait()
        col0 = jnp.zeros((16,), jnp.int32)
        col1 = jnp.full((16,), 1, jnp.int32)

        def ext(j, carry):
            r = lax.iota(jnp.int32, 16) + j * 16
            i0_v[pl.ds(j * 16, 16)] = plsc.load_gather(
                m_v, [r, col0]).astype(jnp.int32)
            i1_v[pl.ds(j * 16, 16)] = plsc.load_gather(
                m_v, [r, col1]).astype(jnp.int32)
            return carry
        lax.fori_loop(0, tok_per_w // 16, ext, 0)
        cpx.wait()
        cp0 = pltpu.async_copy(rows_v, xb_hbm.at[i0_v], sem0)
        cp1 = pltpu.async_copy(rows_v, xb_hbm.at[i1_v], sem1)
        cp0.wait()
        cp1.wait()

    return k(x, meta)


_SUB = 256


def _ffn_body(nb_ref, xb_ref, win_ref, wout_ref, yb_ref):
    e = pl.program_id(0)
    f = pl.program_id(1)
    n = nb_ref[0, e]
    for i in range(C // _SUB):
        @pl.when(jnp.float32(i) < n)
        def _sub():
            xs = xb_ref[pl.ds(i * _SUB, _SUB), :]
            h = jnp.dot(xs, win_ref[0], preferred_element_type=jnp.float32)
            h = jnp.maximum(h, 0.0)
            part = jnp.dot(h, wout_ref[0], preferred_element_type=jnp.float32)

            @pl.when(f == 0)
            def _init():
                yb_ref[pl.ds(i * _SUB, _SUB), :] = part

            @pl.when(f > 0)
            def _acc():
                yb_ref[pl.ds(i * _SUB, _SUB), :] = (
                    yb_ref[pl.ds(i * _SUB, _SUB), :] + part)


def _ffn(nb, xb, w_in, w_out):
    BF = 2048
    nf = D_FF // BF
    return pl.pallas_call(
        _ffn_body,
        grid=(E, nf),
        in_specs=[
            pl.BlockSpec(memory_space=pltpu.SMEM),
            pl.BlockSpec((C, D_MODEL), lambda e, f: (e, 0)),
            pl.BlockSpec((1, D_MODEL, BF), lambda e, f: (e, 0, f)),
            pl.BlockSpec((1, BF, D_MODEL), lambda e, f: (e, f, 0)),
        ],
        out_specs=pl.BlockSpec((C, D_MODEL), lambda e, f: (e, 0)),
        out_shape=jax.ShapeDtypeStruct((EC, D_MODEL), jnp.float32),
    )(nb, xb, w_in, w_out)


def _combine(yb, meta):
    NC, NS = _sc_dims()
    NW = NC * NS
    tok_per_w = T // NW
    bt = 16
    n_chunks = tok_per_w // bt
    mesh = plsc.VectorSubcoreMesh(core_axis_name="c", subcore_axis_name="s")

    @functools.partial(
        pl.kernel,
        mesh=mesh,
        out_type=jax.ShapeDtypeStruct((T, D_MODEL), jnp.float32),
        compiler_params=pltpu.CompilerParams(needs_layout_passes=False),
        scratch_types=[
            pltpu.VMEM((tok_per_w, LANES), jnp.float32),
            pltpu.VMEM((tok_per_w + 16,), jnp.float32),
            pltpu.VMEM((tok_per_w + 16,), jnp.float32),
            pltpu.VMEM((2, bt), jnp.int32),
            pltpu.VMEM((2, bt), jnp.int32),
            pltpu.VMEM((2, bt, D_MODEL), jnp.float32),
            pltpu.VMEM((2, bt, D_MODEL), jnp.float32),
            pltpu.VMEM((2, bt, D_MODEL), jnp.float32),
            pltpu.SemaphoreType.DMA,
            pltpu.SemaphoreType.DMA,
            pltpu.SemaphoreType.DMA,
        ],
    )
    def k(yb_hbm, meta_hbm, out_hbm,
          m_v, g0_v, g1_v, i0_v, i1_v, r0_v, r1_v, o_v, semg0, semg1, semw):
        wid = lax.axis_index("s") * NC + lax.axis_index("c")
        tbase = wid * tok_per_w
        pltpu.sync_copy(meta_hbm.at[pl.ds(tbase, tok_per_w)], m_v)
        col2 = jnp.full((16,), 2, jnp.int32)
        col3 = jnp.full((16,), 3, jnp.int32)
        col4 = jnp.full((16,), 4, jnp.int32)
        col5 = jnp.full((16,), 5, jnp.int32)

        def ext(j, carry):
            r = lax.iota(jnp.int32, 16) + j * 16
            g0_v[pl.ds(j * 16, 16)] = plsc.load_gather(m_v, [r, col4])
            g1_v[pl.ds(j * 16, 16)] = plsc.load_gather(m_v, [r, col5])
            return carry
        lax.fori_loop(0, tok_per_w // 16, ext, 0)

        def issue(ci, p):
            r = lax.iota(jnp.int32, 16) + ci * bt
            i0_v[p, :] = plsc.load_gather(m_v, [r, col2]).astype(jnp.int32)
            i1_v[p, :] = plsc.load_gather(m_v, [r, col3]).astype(jnp.int32)
            cpa = pltpu.async_copy(yb_hbm.at[i0_v.at[p]], r0_v.at[p], semg0)
            cpb = pltpu.async_copy(yb_hbm.at[i1_v.at[p]], r1_v.at[p], semg1)
            return cpa, cpb

        pend = {0: issue(0, 0)}
        wr = {}
        for ci in range(n_chunks):
            p = ci % 2
            cpa, cpb = pend.pop(ci)
            if ci + 1 < n_chunks:
                pend[ci + 1] = issue(ci + 1, 1 - p)
            cpa.wait()
            cpb.wait()
            if ci >= 2:
                wr.pop(ci - 2).wait()

            def row_i(r, c2, ci=ci, p=p):
                a = g0_v[pl.ds(ci * bt + r, 16)][0]
                b = g1_v[pl.ds(ci * bt + r, 16)][0]

                def lane_i(j, c3):
                    for u in range(4):
                        sl = pl.ds(j * 64 + u * 16, 16)
                        o_v[p, r, sl] = a * r0_v[p, r, sl] + b * r1_v[p, r, sl]
                    return c3
                lax.fori_loop(0, D_MODEL // 64, lane_i, 0)
                return c2
            lax.fori_loop(0, bt, row_i, 0)
            wr[ci] = pltpu.async_copy(
                o_v.at[p], out_hbm.at[pl.ds(tbase + ci * bt, bt)], semw)
        for c in sorted(wr):
            wr.pop(c).wait()

    return k(yb, meta)


def kernel(x, w_router, w_in, w_out):
    wr_pad = jnp.zeros((D_MODEL, LANES), jnp.float32).at[:, :E].set(w_router)
    meta, nb_row = _router(x, wr_pad)
    xb = _dispatch(x, meta)
    yb = _ffn(nb_row, xb, w_in, w_out)
    return _combine(yb, meta)


# speedup vs baseline: 2.1227x; 1.0104x over previous
import functools

import jax
import jax.numpy as jnp
from jax import lax
from jax.experimental import pallas as pl
from jax.experimental.pallas import tpu as pltpu
from jax.experimental.pallas import tpu_sc as plsc

E = 8
TOP_K = 2
D_MODEL = 1024
D_FF = 4096
T = 2048
C = 1024
EC = E * C
LANES = 128


def _router_body(x_ref, wr_ref, meta_ref, nb_ref):
    x = x_ref[...]
    wr = wr_ref[...]
    logits = jnp.dot(x, wr, preferred_element_type=jnp.float32)
    col = lax.broadcasted_iota(jnp.int32, (T, LANES), 1).astype(jnp.float32)
    real = col < E
    logits = jnp.where(real, logits, -1e30)
    mx = jnp.max(logits, axis=1, keepdims=True)
    ex = jnp.exp(logits - mx)
    probs = ex / jnp.sum(ex, axis=1, keepdims=True)
    psel = jnp.where(real, probs, -1.0)
    p0 = jnp.max(psel, axis=1, keepdims=True)
    e0 = jnp.min(jnp.where(psel >= p0, col, 128.0), axis=1, keepdims=True)
    psel2 = jnp.where(col == e0, -2.0, psel)
    p1 = jnp.max(psel2, axis=1, keepdims=True)
    e1 = jnp.min(jnp.where(psel2 >= p1, col, 128.0), axis=1, keepdims=True)
    s = p0 + p1
    g0 = p0 / s
    g1 = p1 / s
    oh0 = (col == e0).astype(jnp.float32)
    oh1 = (col == e1).astype(jnp.float32)
    cnt = oh0 + oh1
    ri = lax.broadcasted_iota(jnp.int32, (T, T), 0)
    ci = lax.broadcasted_iota(jnp.int32, (T, T), 1)
    tri = (ri > ci).astype(jnp.float32)
    c1 = jnp.dot(tri, cnt, preferred_element_type=jnp.float32)
    pos0 = jnp.sum(oh0 * c1, axis=1, keepdims=True)
    pos1 = jnp.sum(oh1 * c1, axis=1, keepdims=True)
    slot0 = e0 * C + pos0
    slot1 = e1 * C + pos1
    v0 = pos0 < C
    v1 = pos1 < C
    sslot0 = jnp.where(v0, slot0, float(EC))
    sslot1 = jnp.where(v1, slot1, float(EC))
    cslot0 = jnp.where(v0, slot0, e0 * C)
    cslot1 = jnp.where(v1, slot1, e1 * C)
    gg0 = jnp.where(v0, g0, 0.0)
    gg1 = jnp.where(v1, g1, 0.0)
    meta = jnp.where(col < 1.0, sslot0,
           jnp.where(col < 2.0, sslot1,
           jnp.where(col < 3.0, cslot0,
           jnp.where(col < 4.0, cslot1,
           jnp.where(col < 5.0, gg0, gg1)))))
    meta_ref[...] = meta
    nb_ref[...] = jnp.minimum(jnp.sum(cnt, axis=0, keepdims=True), float(C))


def _router(x, wr_pad):
    return pl.pallas_call(
        _router_body,
        out_shape=[
            jax.ShapeDtypeStruct((T, LANES), jnp.float32),
            jax.ShapeDtypeStruct((1, LANES), jnp.float32),
        ],
    )(x, wr_pad)


def _sc_dims():
    try:
        info = plsc.get_sparse_core_info()
        return info.num_cores, info.num_subcores
    except Exception:
        return 2, 16


def _dispatch(x, meta):
    NC, NS = _sc_dims()
    NW = NC * NS
    tok_per_w = T // NW
    mesh = plsc.VectorSubcoreMesh(core_axis_name="c", subcore_axis_name="s")

    @functools.partial(
        pl.kernel,
        mesh=mesh,
        out_type=jax.ShapeDtypeStruct((EC + 8, D_MODEL), jnp.float32),
        compiler_params=pltpu.CompilerParams(needs_layout_passes=False),
        scratch_types=[
            pltpu.VMEM((tok_per_w, LANES), jnp.float32),
            pltpu.VMEM((tok_per_w,), jnp.int32),
            pltpu.VMEM((tok_per_w,), jnp.int32),
            pltpu.VMEM((tok_per_w, D_MODEL), jnp.float32),
            pltpu.SemaphoreType.DMA,
            pltpu.SemaphoreType.DMA,
        ],
    )
    def k(x_hbm, meta_hbm, xb_hbm, m_v, i0_v, i1_v, rows_v, sem0, sem1):
        wid = lax.axis_index("s") * NC + lax.axis_index("c")
        tbase = wid * tok_per_w
        cpm = pltpu.async_copy(meta_hbm.at[pl.ds(tbase, tok_per_w)], m_v, sem0)
        cpx = pltpu.async_copy(x_hbm.at[pl.ds(tbase, tok_per_w)], rows_v, sem1)
        cpm.wait()
        col0 = jnp.zeros((16,), jnp.int32)
        col1 = jnp.full((16,), 1, jnp.int32)

        def ext(j, carry):
            r = lax.iota(jnp.int32, 16) + j * 16
            i0_v[pl.ds(j * 16, 16)] = plsc.load_gather(
                m_v, [r, col0]).astype(jnp.int32)
            i1_v[pl.ds(j * 16, 16)] = plsc.load_gather(
                m_v, [r, col1]).astype(jnp.int32)
            return carry
        lax.fori_loop(0, tok_per_w // 16, ext, 0)
        cpx.wait()
        cp0 = pltpu.async_copy(rows_v, xb_hbm.at[i0_v], sem0)
        cp1 = pltpu.async_copy(rows_v, xb_hbm.at[i1_v], sem1)
        cp0.wait()
        cp1.wait()

    return k(x, meta)


_SUB = 128


def _ffn_body(nb_ref, xb_ref, win_ref, wout_ref, yb_ref):
    e = pl.program_id(0)
    f = pl.program_id(1)
    n = nb_ref[0, e]
    for i in range(C // _SUB):
        @pl.when(jnp.float32(i * _SUB) < n)
        def _sub():
            xs = xb_ref[pl.ds(i * _SUB, _SUB), :]
            h = jnp.dot(xs, win_ref[0], preferred_element_type=jnp.float32)
            h = jnp.maximum(h, 0.0)
            part = jnp.dot(h, wout_ref[0], preferred_element_type=jnp.float32)

            @pl.when(f == 0)
            def _init():
                yb_ref[pl.ds(i * _SUB, _SUB), :] = part

            @pl.when(f > 0)
            def _acc():
                yb_ref[pl.ds(i * _SUB, _SUB), :] = (
                    yb_ref[pl.ds(i * _SUB, _SUB), :] + part)


def _ffn(nb, xb, w_in, w_out):
    BF = 2048
    nf = D_FF // BF
    return pl.pallas_call(
        _ffn_body,
        grid=(E, nf),
        in_specs=[
            pl.BlockSpec(memory_space=pltpu.SMEM),
            pl.BlockSpec((C, D_MODEL), lambda e, f: (e, 0)),
            pl.BlockSpec((1, D_MODEL, BF), lambda e, f: (e, 0, f)),
            pl.BlockSpec((1, BF, D_MODEL), lambda e, f: (e, f, 0)),
        ],
        out_specs=pl.BlockSpec((C, D_MODEL), lambda e, f: (e, 0)),
        out_shape=jax.ShapeDtypeStruct((EC, D_MODEL), jnp.float32),
    )(nb, xb, w_in, w_out)


def _combine(yb, meta):
    NC, NS = _sc_dims()
    NW = NC * NS
    tok_per_w = T // NW
    bt = 16
    n_chunks = tok_per_w // bt
    mesh = plsc.VectorSubcoreMesh(core_axis_name="c", subcore_axis_name="s")

    @functools.partial(
        pl.kernel,
        mesh=mesh,
        out_type=jax.ShapeDtypeStruct((T, D_MODEL), jnp.float32),
        compiler_params=pltpu.CompilerParams(needs_layout_passes=False),
        scratch_types=[
            pltpu.VMEM((tok_per_w, LANES), jnp.float32),
            pltpu.VMEM((tok_per_w + 16,), jnp.float32),
            pltpu.VMEM((tok_per_w + 16,), jnp.float32),
            pltpu.VMEM((2, bt), jnp.int32),
            pltpu.VMEM((2, bt), jnp.int32),
            pltpu.VMEM((2, bt, D_MODEL), jnp.float32),
            pltpu.VMEM((2, bt, D_MODEL), jnp.float32),
            pltpu.VMEM((2, bt, D_MODEL), jnp.float32),
            pltpu.SemaphoreType.DMA,
            pltpu.SemaphoreType.DMA,
            pltpu.SemaphoreType.DMA,
        ],
    )
    def k(yb_hbm, meta_hbm, out_hbm,
          m_v, g0_v, g1_v, i0_v, i1_v, r0_v, r1_v, o_v, semg0, semg1, semw):
        wid = lax.axis_index("s") * NC + lax.axis_index("c")
        tbase = wid * tok_per_w
        pltpu.sync_copy(meta_hbm.at[pl.ds(tbase, tok_per_w)], m_v)
        col2 = jnp.full((16,), 2, jnp.int32)
        col3 = jnp.full((16,), 3, jnp.int32)
        col4 = jnp.full((16,), 4, jnp.int32)
        col5 = jnp.full((16,), 5, jnp.int32)

        def ext(j, carry):
            r = lax.iota(jnp.int32, 16) + j * 16
            g0_v[pl.ds(j * 16, 16)] = plsc.load_gather(m_v, [r, col4])
            g1_v[pl.ds(j * 16, 16)] = plsc.load_gather(m_v, [r, col5])
            return carry
        lax.fori_loop(0, tok_per_w // 16, ext, 0)

        def issue(ci, p):
            r = lax.iota(jnp.int32, 16) + ci * bt
            i0_v[p, :] = plsc.load_gather(m_v, [r, col2]).astype(jnp.int32)
            i1_v[p, :] = plsc.load_gather(m_v, [r, col3]).astype(jnp.int32)
            cpa = pltpu.async_copy(yb_hbm.at[i0_v.at[p]], r0_v.at[p], semg0)
            cpb = pltpu.async_copy(yb_hbm.at[i1_v.at[p]], r1_v.at[p], semg1)
            return cpa, cpb

        pend = {0: issue(0, 0)}
        wr = {}
        for ci in range(n_chunks):
            p = ci % 2
            cpa, cpb = pend.pop(ci)
            if ci + 1 < n_chunks:
                pend[ci + 1] = issue(ci + 1, 1 - p)
            cpa.wait()
            cpb.wait()
            if ci >= 2:
                wr.pop(ci - 2).wait()

            def row_i(r, c2, ci=ci, p=p):
                a = g0_v[pl.ds(ci * bt + r, 16)][0]
                b = g1_v[pl.ds(ci * bt + r, 16)][0]

                def lane_i(j, c3):
                    for u in range(4):
                        sl = pl.ds(j * 64 + u * 16, 16)
                        o_v[p, r, sl] = a * r0_v[p, r, sl] + b * r1_v[p, r, sl]
                    return c3
                lax.fori_loop(0, D_MODEL // 64, lane_i, 0)
                return c2
            lax.fori_loop(0, bt, row_i, 0)
            wr[ci] = pltpu.async_copy(
                o_v.at[p], out_hbm.at[pl.ds(tbase + ci * bt, bt)], semw)
        for c in sorted(wr):
            wr.pop(c).wait()

    return k(yb, meta)


def kernel(x, w_router, w_in, w_out):
    wr_pad = jnp.zeros((D_MODEL, LANES), jnp.float32).at[:, :E].set(w_router)
    meta, nb_row = _router(x, wr_pad)
    xb = _dispatch(x, meta)
    yb = _ffn(nb_row, xb, w_in, w_out)
    return _combine(yb, meta)
